# baseline (speedup 1.0000x reference)
import dataclasses
import functools

import jax
import jax.numpy as jnp
from jax import lax
from jax.experimental import pallas as pl
from jax.experimental.pallas import tpu as pltpu
from jax.experimental.pallas import tpu_sc as plsc

N = 10000
D = 256
H = 128
E = 160000
P = 100000
T = 2 * P

_EK = 128
_ECHUNK = 1
_NCHUNK = E // (_EK * _ECHUNK)

_PK = 64
_TPAD = 204800
_PCHUNKS = _TPAD // (32 * _PK)

_mesh = plsc.VectorSubcoreMesh(core_axis_name="c", subcore_axis_name="s")
_f32 = jnp.float32

_sc_params = pltpu.CompilerParams()
if "needs_layout_passes" in pltpu.CompilerParams.__dataclass_fields__:
    _sc_params = dataclasses.replace(_sc_params, needs_layout_passes=False)


def _seg_body(compute_deg, xa0, xa1, src3, dst3, *refs):
    if compute_deg:
        (agg0, agg1, degp, srcv0, dstv0, srcv1, dstv1, rows0, rows1,
         deg, acc, sem0, sem1) = refs
    else:
        (agg0, agg1, srcv0, dstv0, srcv1, dstv1, rows0, rows1,
         acc, sem0, sem1) = refs
    rows = rows0
    cid = lax.axis_index("c")
    sid = lax.axis_index("s")

    @pl.loop(0, 104)
    def _(r):
        @pl.loop(0, H, step=16)
        def _(col):
            rows[r, pl.ds(col, 16)] = jnp.zeros((16,), _f32)

    @pl.loop(0, 6)
    def _(k):
        pltpu.sync_copy(rows.at[pl.ds(0, 104)],
                        acc.at[pl.ds(sid * 624 + k * 104, 104)])

    @pl.when(sid == 15)
    def _():
        pltpu.sync_copy(rows.at[pl.ds(0, 16)], acc.at[pl.ds(9984, 16)])

    if compute_deg:
        @pl.when(cid == 0)
        def _():
            @pl.loop(0, N, step=16)
            def _(n):
                deg[pl.ds(n, 16)] = jnp.zeros((16,), _f32)

    plsc.subcore_barrier()

    nfull = _NCHUNK // 16
    nchunk = jnp.where(sid < _NCHUNK - 16 * nfull, nfull + 1, nfull)
    ones16 = jnp.ones((16,), _f32)

    def run(tab, with_deg):
        slots = ((srcv0, dstv0, rows0, sem0), (srcv1, dstv1, rows1, sem1))

        def fetch(k, slot):
            srcv, dstv, rws, sem = slots[slot]
            c = sid + 16 * k
            pltpu.sync_copy(src3.at[c], srcv)
            pltpu.sync_copy(dst3.at[c], dstv)
            for j in range(_ECHUNK):
                pltpu.async_copy(tab.at[srcv.at[j]],
                                 rws.at[pl.ds(j * _EK, _EK)], sem)

        def flush(slot):
            srcv, dstv, rws, sem = slots[slot]
            for j in range(_ECHUNK):
                pltpu.make_async_copy(tab.at[srcv.at[j]],
                                      rws.at[pl.ds(j * _EK, _EK)], sem).wait()
            for j in range(_ECHUNK):
                pltpu.sync_copy(rws.at[pl.ds(j * _EK, _EK)],
                                acc.at[dstv.at[j]], add=True)
            if with_deg:
                for j in range(_ECHUNK):
                    for g in range(_EK // 16):
                        idx = dstv[j, pl.ds(g * 16, 16)]
                        plsc.addupdate_scatter(deg, [idx], ones16)

        fetch(0, 0)

        @pl.loop(0, nchunk // 2)
        def _(h):
            k0 = 2 * h
            fetch(k0 + 1, 1)
            flush(0)

            @pl.when(k0 + 2 < nchunk)
            def _():
                fetch(k0 + 2, 0)

            flush(1)

        @pl.when(nchunk % 2 == 1)
        def _():
            flush(0)

    @pl.when(cid == 0)
    def _():
        run(xa0, compute_deg)

    @pl.when(cid == 1)
    def _():
        run(xa1, False)

    plsc.subcore_barrier()

    def copy_out(agg):
        pltpu.sync_copy(acc.at[pl.ds(sid * 624, 624)],
                        agg.at[pl.ds(sid * 624, 624)])

        @pl.when(sid == 15)
        def _():
            pltpu.sync_copy(acc.at[pl.ds(9984, 16)],
                            agg.at[pl.ds(9984, 16)])

    @pl.when(cid == 0)
    def _():
        copy_out(agg0)
        if compute_deg:
            pltpu.sync_copy(deg, degp.at[pl.ds(sid * N, N)])

    @pl.when(cid == 1)
    def _():
        copy_out(agg1)


def _segment_sum(xa0, xa1, src3, dst3, compute_deg):
    out_type = [jax.ShapeDtypeStruct((N, H), _f32),
                jax.ShapeDtypeStruct((N, H), _f32)]
    scratch = [
        pltpu.VMEM((_ECHUNK, _EK), jnp.int32),
        pltpu.VMEM((_ECHUNK, _EK), jnp.int32),
        pltpu.VMEM((_ECHUNK, _EK), jnp.int32),
        pltpu.VMEM((_ECHUNK, _EK), jnp.int32),
        pltpu.VMEM((_ECHUNK * _EK, H), _f32),
        pltpu.VMEM((_ECHUNK * _EK, H), _f32),
        pltpu.VMEM_SHARED((N, H), _f32),
        pltpu.SemaphoreType.DMA,
        pltpu.SemaphoreType.DMA,
    ]
    if compute_deg:
        out_type = out_type + [jax.ShapeDtypeStruct((16 * N,), _f32)]
        scratch = scratch[:6] + [pltpu.VMEM((N,), _f32)] + scratch[6:]
    return pl.kernel(
        functools.partial(_seg_body, compute_deg),
        out_type=tuple(out_type),
        mesh=_mesh,
        scratch_types=scratch,
        compiler_params=_sc_params,
    )(xa0, xa1, src3, dst3)


def _decode_body(u, v, s2, t2, w2, bvec, out,
                 sidx, tidx, ub0, vb0, ub1, vb1,
                 lg, tots, w2b, bb, su0, sw0, su1, sw1):
    cid = lax.axis_index("c")
    sid = lax.axis_index("s")
    wid = sid * 2 + cid

    pltpu.sync_copy(w2, w2b)
    pltpu.sync_copy(bvec, bb)
    base = wid * _PCHUNKS * _PK
    pltpu.sync_copy(s2.at[pl.ds(base, _PCHUNKS * _PK)], sidx)
    pltpu.sync_copy(t2.at[pl.ds(base, _PCHUNKS * _PK)], tidx)
    wws = [w2b[pl.ds(j * 16, 16)] for j in range(16)]
    bbv = bb[...]
    lane = lax.iota(jnp.int32, 16)

    slots = ((ub0, vb0, su0, sw0), (ub1, vb1, su1, sw1))

    def fetch(i, slot):
        ub, vb, semu, semv = slots[slot]
        pltpu.async_copy(u.at[sidx.at[pl.ds(i * _PK, _PK)]], ub, semu)
        pltpu.async_copy(v.at[tidx.at[pl.ds(i * _PK, _PK)]], vb, semv)

    def compute(i, slot):
        ub, vb, semu, semv = slots[slot]
        pltpu.make_async_copy(
            u.at[sidx.at[pl.ds(i * _PK, _PK)]], ub, semu).wait()
        pltpu.make_async_copy(
            v.at[tidx.at[pl.ds(i * _PK, _PK)]], vb, semv).wait()

        @pl.loop(0, _PK)
        def _(p):
            us = [ub[p, pl.ds(j * 16, 16)] for j in range(16)]
            vs = [vb[p, pl.ds(j * 16, 16)] for j in range(16)]
            accs = [jnp.maximum(us[j] + vs[j], 0.0) * wws[j]
                    for j in range(8)]
            for j in range(8, 16):
                accs[j - 8] = accs[j - 8] + \
                    jnp.maximum(us[j] + vs[j], 0.0) * wws[j]
            a0 = (accs[0] + accs[1]) + (accs[2] + accs[3])
            a1 = (accs[4] + accs[5]) + (accs[6] + accs[7])
            tots[p, :] = (a0 + a1) + bbv

        @pl.loop(0, _PK // 16)
        def _(g):
            row = g * 16 + lane
            acc = plsc.load_gather(tots, [row, jnp.zeros((16,), jnp.int32)])
            for r in range(1, 16):
                acc = acc + plsc.load_gather(
                    tots, [row, jnp.full((16,), r, jnp.int32)])
            lg[i, pl.ds(g * 16, 16)] = acc

    fetch(0, 0)

    @pl.loop(0, _PCHUNKS // 2)
    def _(k):
        i0 = 2 * k
        fetch(i0 + 1, 1)
        compute(i0, 0)

        @pl.when(k < _PCHUNKS // 2 - 1)
        def _():
            fetch(i0 + 2, 0)

        compute(i0 + 1, 1)

    pltpu.sync_copy(lg, out.at[wid])


def _decode(u, v, s2, t2, w2, bvec):
    return pl.kernel(
        _decode_body,
        out_type=jax.ShapeDtypeStruct((32, _PCHUNKS, _PK), _f32),
        mesh=_mesh,
        scratch_types=[
            pltpu.VMEM((_PCHUNKS * _PK,), jnp.int32),
            pltpu.VMEM((_PCHUNKS * _PK,), jnp.int32),
            pltpu.VMEM((_PK, D), _f32),
            pltpu.VMEM((_PK, D), _f32),
            pltpu.VMEM((_PK, D), _f32),
            pltpu.VMEM((_PK, D), _f32),
            pltpu.VMEM((_PCHUNKS, _PK), _f32),
            pltpu.VMEM((_PK, 16), _f32),
            pltpu.VMEM((D,), _f32),
            pltpu.VMEM((16,), _f32),
            pltpu.SemaphoreType.DMA,
            pltpu.SemaphoreType.DMA,
            pltpu.SemaphoreType.DMA,
            pltpu.SemaphoreType.DMA,
        ],
        compiler_params=_sc_params,
    )(u, v, s2, t2, w2, bvec)


_R = 1000


def _dg(a, b):
    return lax.dot_general(a, b, (((1,), (1,)), ((), ())),
                           precision=lax.Precision.HIGHEST,
                           preferred_element_type=_f32)


def _mean_halves(a0, a1, dp):
    deg = lax.dot_general(dp, jnp.ones((16, 1), _f32), (((1,), (0,)), ((), ())),
                          precision=lax.Precision.HIGHEST,
                          preferred_element_type=_f32)
    inv = 1.0 / jnp.maximum(deg, 1.0)
    return a0 * inv, a1 * inv


def _layer1_body(a0, a1, dp, x, wl, wr, b, h0, h1):
    m0, m1 = _mean_halves(a0[...], a1[...], dp[...])
    h = (_dg(m0, wl[:, :H]) + _dg(m1, wl[:, H:])
         + _dg(x[...], wr[...]) + b[...])
    h = jnp.maximum(h, 0.0)
    h0[...] = h[:, :H]
    h1[...] = h[:, H:]


def _layer1(agg0, agg1, dp, x, Wl1, Wr1, bl1):
    grid = (N // _R,)
    bhalf = pl.BlockSpec((_R, H), lambda i: (i, 0))
    return pl.pallas_call(
        _layer1_body,
        grid=grid,
        in_specs=[bhalf, bhalf,
                  pl.BlockSpec((_R, 16), lambda i: (i, 0)),
                  pl.BlockSpec((_R, D), lambda i: (i, 0)),
                  pl.BlockSpec((D, D), lambda i: (0, 0)),
                  pl.BlockSpec((D, D), lambda i: (0, 0)),
                  pl.BlockSpec((1, D), lambda i: (0, 0))],
        out_specs=[bhalf, bhalf],
        out_shape=(jax.ShapeDtypeStruct((N, H), _f32),
                   jax.ShapeDtypeStruct((N, H), _f32)),
    )(agg0, agg1, dp, x, Wl1, Wr1, bl1)


def _layer2_body(a0, a1, dp, h0, h1, wl, wr, bl, wa, wb, be, z, uo, vo):
    m0, m1 = _mean_halves(a0[...], a1[...], dp[...])
    zz = (_dg(m0, wl[:, :H]) + _dg(m1, wl[:, H:])
          + _dg(h0[...], wr[:, :H]) + _dg(h1[...], wr[:, H:])
          + bl[...])
    z[...] = zz
    uo[...] = _dg(zz, wa[...]) + be[...]
    vo[...] = _dg(zz, wb[...])


def _layer2(agg0, agg1, dp, h0, h1, Wl2, Wr2, bl2, Wa, Wb, be1):
    grid = (N // _R,)
    bhalf = pl.BlockSpec((_R, H), lambda i: (i, 0))
    bw = pl.BlockSpec((D, D), lambda i: (0, 0))
    bb = pl.BlockSpec((1, D), lambda i: (0, 0))
    bout = pl.BlockSpec((_R, D), lambda i: (i, 0))
    return pl.pallas_call(
        _layer2_body,
        grid=grid,
        in_specs=[bhalf, bhalf,
                  pl.BlockSpec((_R, 16), lambda i: (i, 0)),
                  bhalf, bhalf, bw, bw, bb, bw, bw, bb],
        out_specs=[bout, bout, bout],
        out_shape=(jax.ShapeDtypeStruct((N, D), _f32),
                   jax.ShapeDtypeStruct((N, D), _f32),
                   jax.ShapeDtypeStruct((N, D), _f32)),
    )(agg0, agg1, dp, h0, h1, Wl2, Wr2, bl2, Wa, Wb, be1)


def kernel(x, edge_index, pos_edge_index, neg_edge_index,
           Wl1, bl1, Wr1, Wl2, bl2, Wr2, We1, be1, We2, be2):
    src3 = edge_index[0].reshape(_NCHUNK, _ECHUNK, _EK)
    dst3 = edge_index[1].reshape(_NCHUNK, _ECHUNK, _EK)

    xa0 = x[:, :H]
    xa1 = x[:, H:]

    agg0, agg1, degp = _segment_sum(xa0, xa1, src3, dst3, True)
    dp = degp.reshape(16, N).T
    h0, h1 = _layer1(agg0, agg1, dp, x, Wl1, Wr1, bl1.reshape(1, D))

    agg0b, agg1b = _segment_sum(h0, h1, src3, dst3, False)
    z, u, v = _layer2(agg0b, agg1b, dp, h0, h1, Wl2, Wr2, bl2.reshape(1, D),
                      We1[:, :D], We1[:, D:], be1.reshape(1, D))

    pad = jnp.zeros((_TPAD - T,), jnp.int32)
    s2 = jnp.concatenate([pos_edge_index[0], neg_edge_index[0], pad])
    t2 = jnp.concatenate([pos_edge_index[1], neg_edge_index[1], pad])
    w2 = We2.reshape(D)
    bvec = jnp.full((16,), be2[0] / 16.0, _f32)

    logits = _decode(u, v, s2, t2, w2, bvec).reshape(_TPAD)
    return (logits[:P], logits[P:T], z)


# speedup vs baseline: 1.0068x; 1.0068x over previous
import dataclasses
import functools

import jax
import jax.numpy as jnp
from jax import lax
from jax.experimental import pallas as pl
from jax.experimental.pallas import tpu as pltpu
from jax.experimental.pallas import tpu_sc as plsc

N = 10000
D = 256
H = 128
E = 160000
P = 100000
T = 2 * P

_EK = 128
_ECHUNK = 1
_NCHUNK = E // (_EK * _ECHUNK)

_PK = 64
_TPAD = 204800
_PCHUNKS = _TPAD // (32 * _PK)

_mesh = plsc.VectorSubcoreMesh(core_axis_name="c", subcore_axis_name="s")
_f32 = jnp.float32

_sc_params = pltpu.CompilerParams()
if "needs_layout_passes" in pltpu.CompilerParams.__dataclass_fields__:
    _sc_params = dataclasses.replace(_sc_params, needs_layout_passes=False)


def _seg_body(compute_deg, xa0, xa1, src3, dst3, *refs):
    if compute_deg:
        (agg0, agg1, degp, srcv0, dstv0, srcv1, dstv1, rows0, rows1,
         deg, acc, sem0, sem1) = refs
    else:
        (agg0, agg1, srcv0, dstv0, srcv1, dstv1, rows0, rows1,
         acc, sem0, sem1) = refs
    rows = rows0
    cid = lax.axis_index("c")
    sid = lax.axis_index("s")

    @pl.loop(0, 104)
    def _(r):
        @pl.loop(0, H, step=16)
        def _(col):
            rows[r, pl.ds(col, 16)] = jnp.zeros((16,), _f32)

    @pl.loop(0, 6)
    def _(k):
        pltpu.sync_copy(rows.at[pl.ds(0, 104)],
                        acc.at[pl.ds(sid * 624 + k * 104, 104)])

    @pl.when(sid == 15)
    def _():
        pltpu.sync_copy(rows.at[pl.ds(0, 16)], acc.at[pl.ds(9984, 16)])

    if compute_deg:
        @pl.when(cid == 0)
        def _():
            @pl.loop(0, N, step=16)
            def _(n):
                deg[pl.ds(n, 16)] = jnp.zeros((16,), _f32)

    plsc.subcore_barrier()

    nfull = _NCHUNK // 16
    nchunk = jnp.where(sid < _NCHUNK - 16 * nfull, nfull + 1, nfull)
    ones16 = jnp.ones((16,), _f32)

    def run(tab, with_deg):
        slots = ((srcv0, dstv0, rows0, sem0), (srcv1, dstv1, rows1, sem1))

        def fetch(k, slot):
            srcv, dstv, rws, sem = slots[slot]
            c = sid + 16 * k
            pltpu.sync_copy(src3.at[c], srcv)
            pltpu.sync_copy(dst3.at[c], dstv)
            for j in range(_ECHUNK):
                pltpu.async_copy(tab.at[srcv.at[j]],
                                 rws.at[pl.ds(j * _EK, _EK)], sem)

        def flush(slot):
            srcv, dstv, rws, sem = slots[slot]
            for j in range(_ECHUNK):
                pltpu.make_async_copy(tab.at[srcv.at[j]],
                                      rws.at[pl.ds(j * _EK, _EK)], sem).wait()
            for j in range(_ECHUNK):
                pltpu.sync_copy(rws.at[pl.ds(j * _EK, _EK)],
                                acc.at[dstv.at[j]], add=True)
            if with_deg:
                for j in range(_ECHUNK):
                    for g in range(_EK // 16):
                        idx = dstv[j, pl.ds(g * 16, 16)]
                        plsc.addupdate_scatter(deg, [idx], ones16)

        fetch(0, 0)

        @pl.loop(0, nchunk // 2)
        def _(h):
            k0 = 2 * h
            fetch(k0 + 1, 1)
            flush(0)

            @pl.when(k0 + 2 < nchunk)
            def _():
                fetch(k0 + 2, 0)

            flush(1)

        @pl.when(nchunk % 2 == 1)
        def _():
            flush(0)

    @pl.when(cid == 0)
    def _():
        run(xa0, compute_deg)

    @pl.when(cid == 1)
    def _():
        run(xa1, False)

    plsc.subcore_barrier()

    def copy_out(agg):
        pltpu.sync_copy(acc.at[pl.ds(sid * 624, 624)],
                        agg.at[pl.ds(sid * 624, 624)])

        @pl.when(sid == 15)
        def _():
            pltpu.sync_copy(acc.at[pl.ds(9984, 16)],
                            agg.at[pl.ds(9984, 16)])

    @pl.when(cid == 0)
    def _():
        copy_out(agg0)
        if compute_deg:
            pltpu.sync_copy(deg, degp.at[pl.ds(sid * N, N)])

    @pl.when(cid == 1)
    def _():
        copy_out(agg1)


def _segment_sum(xa0, xa1, src3, dst3, compute_deg):
    out_type = [jax.ShapeDtypeStruct((N, H), _f32),
                jax.ShapeDtypeStruct((N, H), _f32)]
    scratch = [
        pltpu.VMEM((_ECHUNK, _EK), jnp.int32),
        pltpu.VMEM((_ECHUNK, _EK), jnp.int32),
        pltpu.VMEM((_ECHUNK, _EK), jnp.int32),
        pltpu.VMEM((_ECHUNK, _EK), jnp.int32),
        pltpu.VMEM((_ECHUNK * _EK, H), _f32),
        pltpu.VMEM((_ECHUNK * _EK, H), _f32),
        pltpu.VMEM_SHARED((N, H), _f32),
        pltpu.SemaphoreType.DMA,
        pltpu.SemaphoreType.DMA,
    ]
    if compute_deg:
        out_type = out_type + [jax.ShapeDtypeStruct((16 * N,), _f32)]
        scratch = scratch[:6] + [pltpu.VMEM((N,), _f32)] + scratch[6:]
    return pl.kernel(
        functools.partial(_seg_body, compute_deg),
        out_type=tuple(out_type),
        mesh=_mesh,
        scratch_types=scratch,
        compiler_params=_sc_params,
    )(xa0, xa1, src3, dst3)


def _decode_body(u, v, s2, t2, w2, bvec, out,
                 sidx, tidx, ub0, vb0, ub1, vb1,
                 lg, tots, w2b, bb, su0, sw0, su1, sw1):
    cid = lax.axis_index("c")
    sid = lax.axis_index("s")
    wid = sid * 2 + cid

    pltpu.sync_copy(w2, w2b)
    pltpu.sync_copy(bvec, bb)
    base = wid * _PCHUNKS * _PK
    pltpu.sync_copy(s2.at[pl.ds(base, _PCHUNKS * _PK)], sidx)
    pltpu.sync_copy(t2.at[pl.ds(base, _PCHUNKS * _PK)], tidx)
    wws = [w2b[pl.ds(j * 16, 16)] for j in range(16)]
    bbv = bb[...]
    lane = lax.iota(jnp.int32, 16)

    slots = ((ub0, vb0, su0, sw0), (ub1, vb1, su1, sw1))

    def fetch(i, slot):
        ub, vb, semu, semv = slots[slot]
        pltpu.async_copy(u.at[sidx.at[pl.ds(i * _PK, _PK)]], ub, semu)
        pltpu.async_copy(v.at[tidx.at[pl.ds(i * _PK, _PK)]], vb, semv)

    def compute(i, slot):
        ub, vb, semu, semv = slots[slot]
        pltpu.make_async_copy(
            u.at[sidx.at[pl.ds(i * _PK, _PK)]], ub, semu).wait()
        pltpu.make_async_copy(
            v.at[tidx.at[pl.ds(i * _PK, _PK)]], vb, semv).wait()

        @plsc.parallel_loop(0, _PK)
        def _(p):
            us = [ub[p, pl.ds(j * 16, 16)] for j in range(16)]
            vs = [vb[p, pl.ds(j * 16, 16)] for j in range(16)]
            accs = [jnp.maximum(us[j] + vs[j], 0.0) * wws[j]
                    for j in range(8)]
            for j in range(8, 16):
                accs[j - 8] = accs[j - 8] + \
                    jnp.maximum(us[j] + vs[j], 0.0) * wws[j]
            a0 = (accs[0] + accs[1]) + (accs[2] + accs[3])
            a1 = (accs[4] + accs[5]) + (accs[6] + accs[7])
            tots[p, :] = (a0 + a1) + bbv

        @plsc.parallel_loop(0, _PK // 16)
        def _(g):
            row = g * 16 + lane
            acc = plsc.load_gather(tots, [row, jnp.zeros((16,), jnp.int32)])
            for r in range(1, 16):
                acc = acc + plsc.load_gather(
                    tots, [row, jnp.full((16,), r, jnp.int32)])
            lg[i, pl.ds(g * 16, 16)] = acc

    fetch(0, 0)

    @pl.loop(0, _PCHUNKS // 2)
    def _(k):
        i0 = 2 * k
        fetch(i0 + 1, 1)
        compute(i0, 0)

        @pl.when(k < _PCHUNKS // 2 - 1)
        def _():
            fetch(i0 + 2, 0)

        compute(i0 + 1, 1)

    pltpu.sync_copy(lg, out.at[wid])


def _decode(u, v, s2, t2, w2, bvec):
    return pl.kernel(
        _decode_body,
        out_type=jax.ShapeDtypeStruct((32, _PCHUNKS, _PK), _f32),
        mesh=_mesh,
        scratch_types=[
            pltpu.VMEM((_PCHUNKS * _PK,), jnp.int32),
            pltpu.VMEM((_PCHUNKS * _PK,), jnp.int32),
            pltpu.VMEM((_PK, D), _f32),
            pltpu.VMEM((_PK, D), _f32),
            pltpu.VMEM((_PK, D), _f32),
            pltpu.VMEM((_PK, D), _f32),
            pltpu.VMEM((_PCHUNKS, _PK), _f32),
            pltpu.VMEM((_PK, 16), _f32),
            pltpu.VMEM((D,), _f32),
            pltpu.VMEM((16,), _f32),
            pltpu.SemaphoreType.DMA,
            pltpu.SemaphoreType.DMA,
            pltpu.SemaphoreType.DMA,
            pltpu.SemaphoreType.DMA,
        ],
        compiler_params=_sc_params,
    )(u, v, s2, t2, w2, bvec)


_R = 1000


def _dg(a, b):
    return lax.dot_general(a, b, (((1,), (1,)), ((), ())),
                           precision=lax.Precision.HIGHEST,
                           preferred_element_type=_f32)


def _mean_halves(a0, a1, dp):
    deg = lax.dot_general(dp, jnp.ones((16, 1), _f32), (((1,), (0,)), ((), ())),
                          precision=lax.Precision.HIGHEST,
                          preferred_element_type=_f32)
    inv = 1.0 / jnp.maximum(deg, 1.0)
    return a0 * inv, a1 * inv


def _layer1_body(a0, a1, dp, x, wl, wr, b, h0, h1):
    m0, m1 = _mean_halves(a0[...], a1[...], dp[...])
    h = (_dg(m0, wl[:, :H]) + _dg(m1, wl[:, H:])
         + _dg(x[...], wr[...]) + b[...])
    h = jnp.maximum(h, 0.0)
    h0[...] = h[:, :H]
    h1[...] = h[:, H:]


def _layer1(agg0, agg1, dp, x, Wl1, Wr1, bl1):
    grid = (N // _R,)
    bhalf = pl.BlockSpec((_R, H), lambda i: (i, 0))
    return pl.pallas_call(
        _layer1_body,
        grid=grid,
        in_specs=[bhalf, bhalf,
                  pl.BlockSpec((_R, 16), lambda i: (i, 0)),
                  pl.BlockSpec((_R, D), lambda i: (i, 0)),
                  pl.BlockSpec((D, D), lambda i: (0, 0)),
                  pl.BlockSpec((D, D), lambda i: (0, 0)),
                  pl.BlockSpec((1, D), lambda i: (0, 0))],
        out_specs=[bhalf, bhalf],
        out_shape=(jax.ShapeDtypeStruct((N, H), _f32),
                   jax.ShapeDtypeStruct((N, H), _f32)),
    )(agg0, agg1, dp, x, Wl1, Wr1, bl1)


def _layer2_body(a0, a1, dp, h0, h1, wl, wr, bl, wa, wb, be, z, uo, vo):
    m0, m1 = _mean_halves(a0[...], a1[...], dp[...])
    zz = (_dg(m0, wl[:, :H]) + _dg(m1, wl[:, H:])
          + _dg(h0[...], wr[:, :H]) + _dg(h1[...], wr[:, H:])
          + bl[...])
    z[...] = zz
    uo[...] = _dg(zz, wa[...]) + be[...]
    vo[...] = _dg(zz, wb[...])


def _layer2(agg0, agg1, dp, h0, h1, Wl2, Wr2, bl2, Wa, Wb, be1):
    grid = (N // _R,)
    bhalf = pl.BlockSpec((_R, H), lambda i: (i, 0))
    bw = pl.BlockSpec((D, D), lambda i: (0, 0))
    bb = pl.BlockSpec((1, D), lambda i: (0, 0))
    bout = pl.BlockSpec((_R, D), lambda i: (i, 0))
    return pl.pallas_call(
        _layer2_body,
        grid=grid,
        in_specs=[bhalf, bhalf,
                  pl.BlockSpec((_R, 16), lambda i: (i, 0)),
                  bhalf, bhalf, bw, bw, bb, bw, bw, bb],
        out_specs=[bout, bout, bout],
        out_shape=(jax.ShapeDtypeStruct((N, D), _f32),
                   jax.ShapeDtypeStruct((N, D), _f32),
                   jax.ShapeDtypeStruct((N, D), _f32)),
    )(agg0, agg1, dp, h0, h1, Wl2, Wr2, bl2, Wa, Wb, be1)


def kernel(x, edge_index, pos_edge_index, neg_edge_index,
           Wl1, bl1, Wr1, Wl2, bl2, Wr2, We1, be1, We2, be2):
    src3 = edge_index[0].reshape(_NCHUNK, _ECHUNK, _EK)
    dst3 = edge_index[1].reshape(_NCHUNK, _ECHUNK, _EK)

    xa0 = x[:, :H]
    xa1 = x[:, H:]

    agg0, agg1, degp = _segment_sum(xa0, xa1, src3, dst3, True)
    dp = degp.reshape(16, N).T
    h0, h1 = _layer1(agg0, agg1, dp, x, Wl1, Wr1, bl1.reshape(1, D))

    agg0b, agg1b = _segment_sum(h0, h1, src3, dst3, False)
    z, u, v = _layer2(agg0b, agg1b, dp, h0, h1, Wl2, Wr2, bl2.reshape(1, D),
                      We1[:, :D], We1[:, D:], be1.reshape(1, D))

    pad = jnp.zeros((_TPAD - T,), jnp.int32)
    s2 = jnp.concatenate([pos_edge_index[0], neg_edge_index[0], pad])
    t2 = jnp.concatenate([pos_edge_index[1], neg_edge_index[1], pad])
    w2 = We2.reshape(D)
    bvec = jnp.full((16,), be2[0] / 16.0, _f32)

    logits = _decode(u, v, s2, t2, w2, bvec).reshape(_TPAD)
    return (logits[:P], logits[P:T], z)


# speedup vs baseline: 1.1315x; 1.1239x over previous
import dataclasses
import functools

import jax
import jax.numpy as jnp
from jax import lax
from jax.experimental import pallas as pl
from jax.experimental.pallas import tpu as pltpu
from jax.experimental.pallas import tpu_sc as plsc

N = 10000
D = 256
H = 128
E = 160000
P = 100000
T = 2 * P

_EK = 128
_ECHUNK = 1
_NCHUNK = E // (_EK * _ECHUNK)

_PK = 64
_TPAD = 204800
_PCHUNKS = _TPAD // (32 * _PK)

_mesh = plsc.VectorSubcoreMesh(core_axis_name="c", subcore_axis_name="s")
_f32 = jnp.float32

_sc_params = pltpu.CompilerParams()
if "needs_layout_passes" in pltpu.CompilerParams.__dataclass_fields__:
    _sc_params = dataclasses.replace(_sc_params, needs_layout_passes=False)


def _seg_body(compute_deg, xa0, xa1, src3, dst3, *refs):
    if compute_deg:
        (agg0, agg1, degp, srcv0, dstv0, srcv1, dstv1, rows0, rows1,
         deg, acc, sem0, sem1) = refs
    else:
        (agg0, agg1, srcv0, dstv0, srcv1, dstv1, rows0, rows1,
         acc, sem0, sem1) = refs
    rows = rows0
    cid = lax.axis_index("c")
    sid = lax.axis_index("s")

    @pl.loop(0, 104)
    def _(r):
        @pl.loop(0, H, step=16)
        def _(col):
            rows[r, pl.ds(col, 16)] = jnp.zeros((16,), _f32)

    @pl.loop(0, 6)
    def _(k):
        pltpu.sync_copy(rows.at[pl.ds(0, 104)],
                        acc.at[pl.ds(sid * 624 + k * 104, 104)])

    @pl.when(sid == 15)
    def _():
        pltpu.sync_copy(rows.at[pl.ds(0, 16)], acc.at[pl.ds(9984, 16)])

    if compute_deg:
        @pl.when(cid == 0)
        def _():
            @pl.loop(0, N, step=16)
            def _(n):
                deg[pl.ds(n, 16)] = jnp.zeros((16,), _f32)

    plsc.subcore_barrier()

    nfull = _NCHUNK // 16
    nchunk = jnp.where(sid < _NCHUNK - 16 * nfull, nfull + 1, nfull)
    ones16 = jnp.ones((16,), _f32)

    def run(tab, with_deg):
        slots = ((srcv0, dstv0, rows0, sem0), (srcv1, dstv1, rows1, sem1))

        def fetch(k, slot):
            srcv, dstv, rws, sem = slots[slot]
            c = sid + 16 * k
            pltpu.sync_copy(src3.at[c], srcv)
            pltpu.sync_copy(dst3.at[c], dstv)
            for j in range(_ECHUNK):
                pltpu.async_copy(tab.at[srcv.at[j]],
                                 rws.at[pl.ds(j * _EK, _EK)], sem)

        def flush(slot):
            srcv, dstv, rws, sem = slots[slot]
            for j in range(_ECHUNK):
                pltpu.make_async_copy(tab.at[srcv.at[j]],
                                      rws.at[pl.ds(j * _EK, _EK)], sem).wait()
            for j in range(_ECHUNK):
                pltpu.sync_copy(rws.at[pl.ds(j * _EK, _EK)],
                                acc.at[dstv.at[j]], add=True)
            if with_deg:
                for j in range(_ECHUNK):
                    for g in range(_EK // 16):
                        idx = dstv[j, pl.ds(g * 16, 16)]
                        plsc.addupdate_scatter(deg, [idx], ones16)

        fetch(0, 0)

        @pl.loop(0, nchunk // 2)
        def _(h):
            k0 = 2 * h
            fetch(k0 + 1, 1)
            flush(0)

            @pl.when(k0 + 2 < nchunk)
            def _():
                fetch(k0 + 2, 0)

            flush(1)

        @pl.when(nchunk % 2 == 1)
        def _():
            flush(0)

    @pl.when(cid == 0)
    def _():
        run(xa0, compute_deg)

    @pl.when(cid == 1)
    def _():
        run(xa1, False)

    plsc.subcore_barrier()

    def copy_out(agg):
        pltpu.sync_copy(acc.at[pl.ds(sid * 624, 624)],
                        agg.at[pl.ds(sid * 624, 624)])

        @pl.when(sid == 15)
        def _():
            pltpu.sync_copy(acc.at[pl.ds(9984, 16)],
                            agg.at[pl.ds(9984, 16)])

    @pl.when(cid == 0)
    def _():
        copy_out(agg0)
        if compute_deg:
            pltpu.sync_copy(deg, degp.at[pl.ds(sid * N, N)])

    @pl.when(cid == 1)
    def _():
        copy_out(agg1)


def _segment_sum(xa0, xa1, src3, dst3, compute_deg):
    out_type = [jax.ShapeDtypeStruct((N, H), _f32),
                jax.ShapeDtypeStruct((N, H), _f32)]
    scratch = [
        pltpu.VMEM((_ECHUNK, _EK), jnp.int32),
        pltpu.VMEM((_ECHUNK, _EK), jnp.int32),
        pltpu.VMEM((_ECHUNK, _EK), jnp.int32),
        pltpu.VMEM((_ECHUNK, _EK), jnp.int32),
        pltpu.VMEM((_ECHUNK * _EK, H), _f32),
        pltpu.VMEM((_ECHUNK * _EK, H), _f32),
        pltpu.VMEM_SHARED((N, H), _f32),
        pltpu.SemaphoreType.DMA,
        pltpu.SemaphoreType.DMA,
    ]
    if compute_deg:
        out_type = out_type + [jax.ShapeDtypeStruct((16 * N,), _f32)]
        scratch = scratch[:6] + [pltpu.VMEM((N,), _f32)] + scratch[6:]
    return pl.kernel(
        functools.partial(_seg_body, compute_deg),
        out_type=tuple(out_type),
        mesh=_mesh,
        scratch_types=scratch,
        compiler_params=_sc_params,
    )(xa0, xa1, src3, dst3)


def _decode_body(u, v, s2, t2, w2, bvec, out,
                 sv0, tv0, sv1, tv1, ub0, vb0, ub1, vb1,
                 lg, tots, w2b, bb, su0, sw0, su1, sw1):
    cid = lax.axis_index("c")
    sid = lax.axis_index("s")
    wid = sid * 2 + cid

    pltpu.sync_copy(w2, w2b)
    pltpu.sync_copy(bvec, bb)
    base = wid * _PCHUNKS * _PK
    wws = [w2b[pl.ds(j * 16, 16)] for j in range(16)]
    bbv = bb[...]
    lane = lax.iota(jnp.int32, 16)

    slots = ((sv0, tv0, ub0, vb0, su0, sw0),
             (sv1, tv1, ub1, vb1, su1, sw1))

    def fetch(i, slot):
        sv, tv, ub, vb, semu, semv = slots[slot]
        pltpu.sync_copy(s2.at[pl.ds(base + i * _PK, _PK)], sv)
        pltpu.sync_copy(t2.at[pl.ds(base + i * _PK, _PK)], tv)
        pltpu.async_copy(u.at[sv], ub, semu)
        pltpu.async_copy(v.at[tv], vb, semv)

    def compute(i, slot):
        sv, tv, ub, vb, semu, semv = slots[slot]
        pltpu.make_async_copy(u.at[sv], ub, semu).wait()
        pltpu.make_async_copy(v.at[tv], vb, semv).wait()

        @plsc.parallel_loop(0, _PK)
        def _(p):
            us = [ub[p, pl.ds(j * 16, 16)] for j in range(16)]
            vs = [vb[p, pl.ds(j * 16, 16)] for j in range(16)]
            accs = [jnp.maximum(us[j] + vs[j], 0.0) * wws[j]
                    for j in range(8)]
            for j in range(8, 16):
                accs[j - 8] = accs[j - 8] + \
                    jnp.maximum(us[j] + vs[j], 0.0) * wws[j]
            a0 = (accs[0] + accs[1]) + (accs[2] + accs[3])
            a1 = (accs[4] + accs[5]) + (accs[6] + accs[7])
            tots[p, :] = (a0 + a1) + bbv

        @plsc.parallel_loop(0, _PK // 16)
        def _(g):
            row = g * 16 + lane
            acc = plsc.load_gather(tots, [row, jnp.zeros((16,), jnp.int32)])
            for r in range(1, 16):
                acc = acc + plsc.load_gather(
                    tots, [row, jnp.full((16,), r, jnp.int32)])
            lg[i, pl.ds(g * 16, 16)] = acc

    fetch(0, 0)

    @pl.loop(0, _PCHUNKS // 2)
    def _(k):
        i0 = 2 * k
        fetch(i0 + 1, 1)
        compute(i0, 0)

        @pl.when(k < _PCHUNKS // 2 - 1)
        def _():
            fetch(i0 + 2, 0)

        compute(i0 + 1, 1)

    pltpu.sync_copy(lg, out.at[wid])


def _decode(u, v, s2, t2, w2, bvec):
    return pl.kernel(
        _decode_body,
        out_type=jax.ShapeDtypeStruct((32, _PCHUNKS, _PK), _f32),
        mesh=_mesh,
        scratch_types=[
            pltpu.VMEM((_PK,), jnp.int32),
            pltpu.VMEM((_PK,), jnp.int32),
            pltpu.VMEM((_PK,), jnp.int32),
            pltpu.VMEM((_PK,), jnp.int32),
            pltpu.VMEM((_PK, D), _f32),
            pltpu.VMEM((_PK, D), _f32),
            pltpu.VMEM((_PK, D), _f32),
            pltpu.VMEM((_PK, D), _f32),
            pltpu.VMEM((_PCHUNKS, _PK), _f32),
            pltpu.VMEM((_PK, 16), _f32),
            pltpu.VMEM((D,), _f32),
            pltpu.VMEM((16,), _f32),
            pltpu.SemaphoreType.DMA,
            pltpu.SemaphoreType.DMA,
            pltpu.SemaphoreType.DMA,
            pltpu.SemaphoreType.DMA,
        ],
        compiler_params=_sc_params,
    )(u, v, s2, t2, w2, bvec)


_R = 1000


def _dg(a, b):
    return lax.dot_general(a, b, (((1,), (1,)), ((), ())),
                           preferred_element_type=_f32)


def _mean_halves(a0, a1, dp):
    deg = lax.dot_general(dp, jnp.ones((16, 1), _f32), (((1,), (0,)), ((), ())),
                          precision=lax.Precision.HIGHEST,
                          preferred_element_type=_f32)
    inv = 1.0 / jnp.maximum(deg, 1.0)
    return a0 * inv, a1 * inv


def _layer1_body(a0, a1, dp, x, wl, wr, b, h0, h1):
    m0, m1 = _mean_halves(a0[...], a1[...], dp[...])
    h = (_dg(m0, wl[:, :H]) + _dg(m1, wl[:, H:])
         + _dg(x[...], wr[...]) + b[...])
    h = jnp.maximum(h, 0.0)
    h0[...] = h[:, :H]
    h1[...] = h[:, H:]


def _layer1(agg0, agg1, dp, x, Wl1, Wr1, bl1):
    grid = (N // _R,)
    bhalf = pl.BlockSpec((_R, H), lambda i: (i, 0))
    return pl.pallas_call(
        _layer1_body,
        grid=grid,
        in_specs=[bhalf, bhalf,
                  pl.BlockSpec((_R, 16), lambda i: (i, 0)),
                  pl.BlockSpec((_R, D), lambda i: (i, 0)),
                  pl.BlockSpec((D, D), lambda i: (0, 0)),
                  pl.BlockSpec((D, D), lambda i: (0, 0)),
                  pl.BlockSpec((1, D), lambda i: (0, 0))],
        out_specs=[bhalf, bhalf],
        out_shape=(jax.ShapeDtypeStruct((N, H), _f32),
                   jax.ShapeDtypeStruct((N, H), _f32)),
    )(agg0, agg1, dp, x, Wl1, Wr1, bl1)


def _layer2_body(a0, a1, dp, h0, h1, wl, wr, bl, wa, wb, be, z, uo, vo):
    m0, m1 = _mean_halves(a0[...], a1[...], dp[...])
    zz = (_dg(m0, wl[:, :H]) + _dg(m1, wl[:, H:])
          + _dg(h0[...], wr[:, :H]) + _dg(h1[...], wr[:, H:])
          + bl[...])
    z[...] = zz
    uo[...] = _dg(zz, wa[...]) + be[...]
    vo[...] = _dg(zz, wb[...])


def _layer2(agg0, agg1, dp, h0, h1, Wl2, Wr2, bl2, Wa, Wb, be1):
    grid = (N // _R,)
    bhalf = pl.BlockSpec((_R, H), lambda i: (i, 0))
    bw = pl.BlockSpec((D, D), lambda i: (0, 0))
    bb = pl.BlockSpec((1, D), lambda i: (0, 0))
    bout = pl.BlockSpec((_R, D), lambda i: (i, 0))
    return pl.pallas_call(
        _layer2_body,
        grid=grid,
        in_specs=[bhalf, bhalf,
                  pl.BlockSpec((_R, 16), lambda i: (i, 0)),
                  bhalf, bhalf, bw, bw, bb, bw, bw, bb],
        out_specs=[bout, bout, bout],
        out_shape=(jax.ShapeDtypeStruct((N, D), _f32),
                   jax.ShapeDtypeStruct((N, D), _f32),
                   jax.ShapeDtypeStruct((N, D), _f32)),
    )(agg0, agg1, dp, h0, h1, Wl2, Wr2, bl2, Wa, Wb, be1)


def kernel(x, edge_index, pos_edge_index, neg_edge_index,
           Wl1, bl1, Wr1, Wl2, bl2, Wr2, We1, be1, We2, be2):
    src3 = edge_index[0].reshape(_NCHUNK, _ECHUNK, _EK)
    dst3 = edge_index[1].reshape(_NCHUNK, _ECHUNK, _EK)

    xa0 = x[:, :H]
    xa1 = x[:, H:]

    agg0, agg1, degp = _segment_sum(xa0, xa1, src3, dst3, True)
    dp = degp.reshape(16, N).T
    h0, h1 = _layer1(agg0, agg1, dp, x, Wl1, Wr1, bl1.reshape(1, D))

    agg0b, agg1b = _segment_sum(h0, h1, src3, dst3, False)
    z, u, v = _layer2(agg0b, agg1b, dp, h0, h1, Wl2, Wr2, bl2.reshape(1, D),
                      We1[:, :D], We1[:, D:], be1.reshape(1, D))

    pad = jnp.zeros((_TPAD - T,), jnp.int32)
    s2 = jnp.concatenate([pos_edge_index[0], neg_edge_index[0], pad])
    t2 = jnp.concatenate([pos_edge_index[1], neg_edge_index[1], pad])
    w2 = We2.reshape(D)
    bvec = jnp.full((16,), be2[0] / 16.0, _f32)

    logits = _decode(u, v, s2, t2, w2, bvec).reshape(_TPAD)
    return (logits[:P], logits[P:T], z)
